# Initial kernel scaffold; baseline (speedup 1.0000x reference)
import jax
import jax.numpy as jnp
from jax.experimental import pallas as pl


def kernel(p, W_cp_feat, W_cp_dir, W_pool_pos, W_fc_pos, Wb_d0, Wb_fc0, Wb_d1, Wb_fc1, Wb_sc, W_pools, W_act_c, W_fc_c):
    raise NotImplementedError("write your pallas kernel here")


# speedup vs baseline: 6.3952x; 6.3952x over previous
import functools

import jax
import jax.numpy as jnp
from jax.experimental import pallas as pl
from jax.experimental.pallas import tpu as pltpu

EPS = 1e-6
KNN = 20
N = 2048
TILE = 256
NT = N // TILE
NEG_FILL = -3.0e38


def _bf(x):
    return x.astype(jnp.bfloat16)


def _dot(a, b, dims):
    return jax.lax.dot_general(_bf(a), _bf(b), (dims, ((), ())),
                               preferred_element_type=jnp.float32)


def _sum3(a0, a1, a2):
    return (a0 + a1) + a2


def _edge_kernel(xnt_ref, xtn_ref, wf_ref, wd_ref, wp4_ref, wfc_ref,
                 out_ref, y4_ref):
    t = pl.program_id(1)
    xs = xnt_ref[0]
    xt = xnt_ref[0, pl.ds(t * TILE, TILE), :]
    xtt = xtn_ref[0, :, pl.ds(t * TILE, TILE)]

    dott = _dot(xs, xt, ((1,), (1,)))
    inner = -2.0 * dott
    xx_all = _sum3(xs[:, 0:1] * xs[:, 0:1], xs[:, 1:2] * xs[:, 1:2],
                   xs[:, 2:3] * xs[:, 2:3])
    xx_t = _sum3(xtt[0:1] * xtt[0:1], xtt[1:2] * xtt[1:2],
                 xtt[2:3] * xtt[2:3])
    P = ((-xx_t) - inner) - xx_all

    iota0 = jax.lax.broadcasted_iota(jnp.int32, (N, TILE), 0)
    xcol = [xs[:, j:j + 1] for j in range(3)]

    wf = wf_ref[...]
    wd = wd_ref[...]
    wfb = [_bf(wf[:, j:j + 1]).astype(jnp.float32) for j in range(3)]
    wdb = [_bf(wd[:, j:j + 1]).astype(jnp.float32) for j in range(3)]
    xtr = [xtt[j:j + 1, :] for j in range(3)]

    for k in range(KNN):
        m = jnp.max(P, axis=0, keepdims=True)
        idx = jnp.min(jnp.where(P == m, iota0, N), axis=0, keepdims=True)
        oh = iota0 == idx
        nbr = [jnp.max(jnp.where(oh, xcol[j], NEG_FILL), axis=0, keepdims=True)
               for j in range(3)]
        P = jnp.where(oh, NEG_FILL, P)

        c0 = [nbr[j] - xtr[j] for j in range(3)]
        c1 = xtr
        c2 = [nbr[(j + 1) % 3] * xtr[(j + 2) % 3]
              - nbr[(j + 2) % 3] * xtr[(j + 1) % 3] for j in range(3)]
        cb = [[_bf(c0[j]).astype(jnp.float32), _bf(c1[j]).astype(jnp.float32),
               _bf(c2[j]).astype(jnp.float32)] for j in range(3)]
        p = [_sum3(wfb[0] * cb[j][0], wfb[1] * cb[j][1], wfb[2] * cb[j][2])
             for j in range(3)]
        d = [_sum3(wdb[0] * cb[j][0], wdb[1] * cb[j][1], wdb[2] * cb[j][2])
             for j in range(3)]
        dotv = _sum3(p[0] * d[0], p[1] * d[1], p[2] * d[2])
        dns = _sum3(d[0] * d[0], d[1] * d[1], d[2] * d[2])
        q = dotv / (dns + EPS)
        msk = dotv >= 0.0
        for j in range(3):
            yj = jnp.where(msk, p[j], p[j] - q * d[j])
            y4_ref[j, pl.ds((k % 4) * 64, 64), pl.ds((k // 4) * TILE, TILE)] = yj

    wp4 = wp4_ref[...]
    best = jnp.full((64, TILE), NEG_FILL, jnp.float32)
    bz = [jnp.zeros((64, TILE), jnp.float32) for _ in range(3)]
    d2 = [_dot(wp4, y4_ref[j], ((1,), (0,))) for j in range(3)]
    for k in range(KNN):
        rs, cs = (k % 4) * 64, (k // 4) * TILE
        yk = [y4_ref[j, pl.ds(rs, 64), pl.ds(cs, TILE)] for j in range(3)]
        d2k = [d2[j][rs:rs + 64, cs:cs + TILE] for j in range(3)]
        dot2 = _sum3(yk[0] * d2k[0], yk[1] * d2k[1], yk[2] * d2k[2])
        upd = dot2 > best
        best = jnp.where(upd, dot2, best)
        bz = [jnp.where(upd, yk[j], bz[j]) for j in range(3)]

    wfc = wfc_ref[...]
    for j in range(3):
        out_ref[0, j] = _dot(wfc, bz[j], ((1,), (0,)))


def _lin(xs, W):
    return [_dot(W, x, ((1,), (0,))) for x in xs]


def _lrelu(xs, W):
    d = _lin(xs, W)
    dotv = _sum3(xs[0] * d[0], xs[1] * d[1], xs[2] * d[2])
    dns = _sum3(d[0] * d[0], d[1] * d[1], d[2] * d[2])
    q = dotv / (dns + EPS)
    msk = dotv >= 0.0
    return [jnp.where(msk, xs[j], xs[j] - q * d[j]) for j in range(3)]


def _block(xs, Wd0, Wfc0, Wd1, Wfc1, Wsc):
    net = _lin(_lrelu(xs, Wd0), Wfc0)
    dx = _lin(_lrelu(net, Wd1), Wfc1)
    sc = _lin(xs, Wsc)
    return [sc[j] + dx[j] for j in range(3)]


def _pool(xs, W):
    d = _lin(xs, W)
    dotv = _sum3(xs[0] * d[0], xs[1] * d[1], xs[2] * d[2])
    C = dotv.shape[0]
    iota1 = jax.lax.broadcasted_iota(jnp.int32, (C, N), 1)
    m = jnp.max(dotv, axis=1, keepdims=True)
    idx = jnp.min(jnp.where(dotv == m, iota1, N), axis=1, keepdims=True)
    oh = iota1 == idx
    return [jnp.max(jnp.where(oh, xs[j], NEG_FILL), axis=1, keepdims=True)
            for j in range(3)]


def _trunk_kernel(x_ref, wbd0_ref, wbfc0_ref, wbd1_ref, wbfc1_ref, wbsc_ref,
                  wpools_ref, wact_ref, wfcc_ref, out_ref):
    net = [x_ref[0, j] for j in range(3)]
    net = _block(net, wbd0_ref[0], wbfc0_ref[0], wbd1_ref[0],
                 wbfc1_ref[0], wbsc_ref[0])
    for i in range(1, 5):
        pld = _pool(net, wpools_ref[i - 1])
        cat = [jnp.concatenate(
            [net[j], jnp.broadcast_to(pld[j], (128, N))], axis=0)
            for j in range(3)]
        net = _block(cat, wbd0_ref[i], wbfc0_ref[i], wbd1_ref[i],
                     wbfc1_ref[i], wbsc_ref[i])
    pld = _pool(net, wpools_ref[4])
    ys = [jnp.transpose(pld[j]) for j in range(3)]
    d = [_dot(ys[j], wact_ref[...], ((1,), (1,))) for j in range(3)]
    dotv = _sum3(ys[0] * d[0], ys[1] * d[1], ys[2] * d[2])
    dns = _sum3(d[0] * d[0], d[1] * d[1], d[2] * d[2])
    q = dotv / (dns + EPS)
    msk = dotv >= 0.0
    out_ref[...] = jnp.zeros((1, 8, 128), jnp.float32)
    for j in range(3):
        yj = jnp.where(msk, ys[j], ys[j] - q * d[j])
        out_ref[0, j:j + 1, :] = _dot(yj, wfcc_ref[...], ((1,), (1,)))


def kernel(p, W_cp_feat, W_cp_dir, W_pool_pos, W_fc_pos, Wb_d0, Wb_fc0,
           Wb_d1, Wb_fc1, Wb_sc, W_pools, W_act_c, W_fc_c):
    B = p.shape[0]
    p_tn = jnp.transpose(p, (0, 2, 1))
    wp4 = jax.scipy.linalg.block_diag(*([W_pool_pos] * 4))

    x_feat = pl.pallas_call(
        _edge_kernel,
        grid=(B, NT),
        in_specs=[
            pl.BlockSpec((1, N, 3), lambda b, t: (b, 0, 0)),
            pl.BlockSpec((1, 3, N), lambda b, t: (b, 0, 0)),
            pl.BlockSpec((64, 3), lambda b, t: (0, 0)),
            pl.BlockSpec((64, 3), lambda b, t: (0, 0)),
            pl.BlockSpec((256, 256), lambda b, t: (0, 0)),
            pl.BlockSpec((256, 64), lambda b, t: (0, 0)),
        ],
        out_specs=pl.BlockSpec((1, 3, 256, TILE), lambda b, t: (b, 0, 0, t)),
        out_shape=jax.ShapeDtypeStruct((B, 3, 256, N), jnp.float32),
        scratch_shapes=[pltpu.VMEM((3, 256, 5 * TILE), jnp.float32)],
    )(p, p_tn, W_cp_feat, W_cp_dir, wp4, W_fc_pos)

    out = pl.pallas_call(
        _trunk_kernel,
        grid=(B,),
        in_specs=[
            pl.BlockSpec((1, 3, 256, N), lambda b: (b, 0, 0, 0)),
            pl.BlockSpec((5, 256, 256), lambda b: (0, 0, 0)),
            pl.BlockSpec((5, 128, 256), lambda b: (0, 0, 0)),
            pl.BlockSpec((5, 128, 128), lambda b: (0, 0, 0)),
            pl.BlockSpec((5, 128, 128), lambda b: (0, 0, 0)),
            pl.BlockSpec((5, 128, 256), lambda b: (0, 0, 0)),
            pl.BlockSpec((5, 128, 128), lambda b: (0, 0, 0)),
            pl.BlockSpec((128, 128), lambda b: (0, 0)),
            pl.BlockSpec((128, 128), lambda b: (0, 0)),
        ],
        out_specs=pl.BlockSpec((1, 8, 128), lambda b: (b, 0, 0)),
        out_shape=jax.ShapeDtypeStruct((B, 8, 128), jnp.float32),
    )(x_feat, Wb_d0, Wb_fc0, Wb_d1, Wb_fc1, Wb_sc, W_pools, W_act_c, W_fc_c)

    return jnp.transpose(out[:, :3, :], (0, 2, 1))


# speedup vs baseline: 10.0468x; 1.5710x over previous
import functools

import jax
import jax.numpy as jnp
from jax.experimental import pallas as pl
from jax.experimental.pallas import tpu as pltpu

EPS = 1e-6
KNN = 20
N = 2048
TILE = 256
NT = N // TILE
NEG_FILL = -3.0e38


def _bf(x):
    return x.astype(jnp.bfloat16)


def _dot(a, b, dims):
    return jax.lax.dot_general(_bf(a), _bf(b), (dims, ((), ())),
                               preferred_element_type=jnp.float32)


def _sum3(a0, a1, a2):
    return (a0 + a1) + a2


def _edge_kernel(xnt_ref, xtn_ref, xseg_ref, wf_ref, wd_ref, wp4_ref, wfc_ref,
                 out_ref, y4_ref):
    t = pl.program_id(1)
    xs = xnt_ref[0]
    xt = xnt_ref[0, pl.ds(t * TILE, TILE), :]
    xtt = xtn_ref[0, :, pl.ds(t * TILE, TILE)]
    xseg = xseg_ref[0]

    dott = _dot(xs, xt, ((1,), (1,)))
    inner = -2.0 * dott
    xx_all = _sum3(xs[:, 0:1] * xs[:, 0:1], xs[:, 1:2] * xs[:, 1:2],
                   xs[:, 2:3] * xs[:, 2:3])
    xx_t = _sum3(xtt[0:1] * xtt[0:1], xtt[1:2] * xtt[1:2],
                 xtt[2:3] * xtt[2:3])
    P = ((-xx_t) - inner) - xx_all

    iota0 = jax.lax.broadcasted_iota(jnp.int32, (N, TILE), 0)

    wf = wf_ref[...]
    wd = wd_ref[...]
    wfb = [_bf(wf[:, j:j + 1]).astype(jnp.float32) for j in range(3)]
    wdb = [_bf(wd[:, j:j + 1]).astype(jnp.float32) for j in range(3)]
    xtr = [xtt[j:j + 1, :] for j in range(3)]

    for k in range(KNN):
        m = jnp.max(P, axis=0, keepdims=True)
        idx = jnp.min(jnp.where(P == m, iota0, N), axis=0, keepdims=True)
        oh = iota0 == idx
        ohf = jnp.where(oh, 1.0, 0.0)
        gt = _dot(xseg, ohf, ((1,), (0,)))
        nbr = [(gt[j:j + 1] + gt[3 + j:4 + j]) + gt[6 + j:7 + j]
               for j in range(3)]
        P = jnp.where(oh, NEG_FILL, P)

        c0 = [nbr[j] - xtr[j] for j in range(3)]
        c1 = xtr
        c2 = [nbr[(j + 1) % 3] * xtr[(j + 2) % 3]
              - nbr[(j + 2) % 3] * xtr[(j + 1) % 3] for j in range(3)]
        cb = [[_bf(c0[j]).astype(jnp.float32), _bf(c1[j]).astype(jnp.float32),
               _bf(c2[j]).astype(jnp.float32)] for j in range(3)]
        p = [_sum3(wfb[0] * cb[j][0], wfb[1] * cb[j][1], wfb[2] * cb[j][2])
             for j in range(3)]
        d = [_sum3(wdb[0] * cb[j][0], wdb[1] * cb[j][1], wdb[2] * cb[j][2])
             for j in range(3)]
        dotv = _sum3(p[0] * d[0], p[1] * d[1], p[2] * d[2])
        dns = _sum3(d[0] * d[0], d[1] * d[1], d[2] * d[2])
        q = dotv / (dns + EPS)
        msk = dotv >= 0.0
        for j in range(3):
            yj = jnp.where(msk, p[j], p[j] - q * d[j])
            y4_ref[j, pl.ds((k % 4) * 64, 64), pl.ds((k // 4) * TILE, TILE)] = yj

    wp4 = wp4_ref[...]
    best = jnp.full((64, TILE), NEG_FILL, jnp.float32)
    bz = [jnp.zeros((64, TILE), jnp.float32) for _ in range(3)]
    d2 = [_dot(wp4, y4_ref[j], ((1,), (0,))) for j in range(3)]
    for k in range(KNN):
        rs, cs = (k % 4) * 64, (k // 4) * TILE
        yk = [y4_ref[j, pl.ds(rs, 64), pl.ds(cs, TILE)] for j in range(3)]
        d2k = [d2[j][rs:rs + 64, cs:cs + TILE] for j in range(3)]
        dot2 = _sum3(yk[0] * d2k[0], yk[1] * d2k[1], yk[2] * d2k[2])
        upd = dot2 > best
        best = jnp.where(upd, dot2, best)
        bz = [jnp.where(upd, yk[j], bz[j]) for j in range(3)]

    wfc = wfc_ref[...]
    for j in range(3):
        out_ref[0, j] = _dot(wfc, bz[j], ((1,), (0,)))


def _lin(xs, W):
    return [_dot(W, x, ((1,), (0,))) for x in xs]


def _lrelu(xs, W):
    d = _lin(xs, W)
    dotv = _sum3(xs[0] * d[0], xs[1] * d[1], xs[2] * d[2])
    dns = _sum3(d[0] * d[0], d[1] * d[1], d[2] * d[2])
    q = dotv / (dns + EPS)
    msk = dotv >= 0.0
    return [jnp.where(msk, xs[j], xs[j] - q * d[j]) for j in range(3)]


def _block(xs, Wd0, Wfc0, Wd1, Wfc1, Wsc):
    net = _lin(_lrelu(xs, Wd0), Wfc0)
    dx = _lin(_lrelu(net, Wd1), Wfc1)
    sc = _lin(xs, Wsc)
    return [sc[j] + dx[j] for j in range(3)]


def _pool(xs, W):
    d = _lin(xs, W)
    dotv = _sum3(xs[0] * d[0], xs[1] * d[1], xs[2] * d[2])
    C = dotv.shape[0]
    iota1 = jax.lax.broadcasted_iota(jnp.int32, (C, N), 1)
    m = jnp.max(dotv, axis=1, keepdims=True)
    idx = jnp.min(jnp.where(dotv == m, iota1, N), axis=1, keepdims=True)
    oh = iota1 == idx
    return [jnp.max(jnp.where(oh, xs[j], NEG_FILL), axis=1, keepdims=True)
            for j in range(3)]


def _trunk_kernel(x_ref, wbd0_ref, wbfc0_ref, wbd1_ref, wbfc1_ref, wbsc_ref,
                  wpools_ref, wact_ref, wfcc_ref, out_ref):
    net = [x_ref[0, j] for j in range(3)]
    net = _block(net, wbd0_ref[0], wbfc0_ref[0], wbd1_ref[0],
                 wbfc1_ref[0], wbsc_ref[0])
    for i in range(1, 5):
        pld = _pool(net, wpools_ref[i - 1])
        cat = [jnp.concatenate(
            [net[j], jnp.broadcast_to(pld[j], (128, N))], axis=0)
            for j in range(3)]
        net = _block(cat, wbd0_ref[i], wbfc0_ref[i], wbd1_ref[i],
                     wbfc1_ref[i], wbsc_ref[i])
    pld = _pool(net, wpools_ref[4])
    ys = [jnp.transpose(pld[j]) for j in range(3)]
    d = [_dot(ys[j], wact_ref[...], ((1,), (1,))) for j in range(3)]
    dotv = _sum3(ys[0] * d[0], ys[1] * d[1], ys[2] * d[2])
    dns = _sum3(d[0] * d[0], d[1] * d[1], d[2] * d[2])
    q = dotv / (dns + EPS)
    msk = dotv >= 0.0
    out_ref[...] = jnp.zeros((1, 8, 128), jnp.float32)
    for j in range(3):
        yj = jnp.where(msk, ys[j], ys[j] - q * d[j])
        out_ref[0, j:j + 1, :] = _dot(yj, wfcc_ref[...], ((1,), (1,)))


def kernel(p, W_cp_feat, W_cp_dir, W_pool_pos, W_fc_pos, Wb_d0, Wb_fc0,
           Wb_d1, Wb_fc1, Wb_sc, W_pools, W_act_c, W_fc_c):
    B = p.shape[0]
    p_tn = jnp.transpose(p, (0, 2, 1))
    wp4 = jax.scipy.linalg.block_diag(*([W_pool_pos] * 4))
    segs, r = [], p_tn
    for _ in range(3):
        b16 = r.astype(jnp.bfloat16).astype(jnp.float32)
        segs.append(b16)
        r = r - b16
    xseg = jnp.concatenate(segs + [jnp.zeros_like(segs[0][:, :1, :])] * 7,
                           axis=1)

    x_feat = pl.pallas_call(
        _edge_kernel,
        grid=(B, NT),
        in_specs=[
            pl.BlockSpec((1, N, 3), lambda b, t: (b, 0, 0)),
            pl.BlockSpec((1, 3, N), lambda b, t: (b, 0, 0)),
            pl.BlockSpec((1, 16, N), lambda b, t: (b, 0, 0)),
            pl.BlockSpec((64, 3), lambda b, t: (0, 0)),
            pl.BlockSpec((64, 3), lambda b, t: (0, 0)),
            pl.BlockSpec((256, 256), lambda b, t: (0, 0)),
            pl.BlockSpec((256, 64), lambda b, t: (0, 0)),
        ],
        out_specs=pl.BlockSpec((1, 3, 256, TILE), lambda b, t: (b, 0, 0, t)),
        out_shape=jax.ShapeDtypeStruct((B, 3, 256, N), jnp.float32),
        scratch_shapes=[pltpu.VMEM((3, 256, 5 * TILE), jnp.float32)],
    )(p, p_tn, xseg, W_cp_feat, W_cp_dir, wp4, W_fc_pos)

    out = pl.pallas_call(
        _trunk_kernel,
        grid=(B,),
        in_specs=[
            pl.BlockSpec((1, 3, 256, N), lambda b: (b, 0, 0, 0)),
            pl.BlockSpec((5, 256, 256), lambda b: (0, 0, 0)),
            pl.BlockSpec((5, 128, 256), lambda b: (0, 0, 0)),
            pl.BlockSpec((5, 128, 128), lambda b: (0, 0, 0)),
            pl.BlockSpec((5, 128, 128), lambda b: (0, 0, 0)),
            pl.BlockSpec((5, 128, 256), lambda b: (0, 0, 0)),
            pl.BlockSpec((5, 128, 128), lambda b: (0, 0, 0)),
            pl.BlockSpec((128, 128), lambda b: (0, 0)),
            pl.BlockSpec((128, 128), lambda b: (0, 0)),
        ],
        out_specs=pl.BlockSpec((1, 8, 128), lambda b: (b, 0, 0)),
        out_shape=jax.ShapeDtypeStruct((B, 8, 128), jnp.float32),
    )(x_feat, Wb_d0, Wb_fc0, Wb_d1, Wb_fc1, Wb_sc, W_pools, W_act_c, W_fc_c)

    return jnp.transpose(out[:, :3, :], (0, 2, 1))
